# baseline (speedup 1.0000x reference)
import functools
import math

import jax
import jax.numpy as jnp
from jax import lax
from jax.experimental import pallas as pl
from jax.experimental.pallas import tpu as pltpu
from jax.experimental.pallas import tpu_sc as plsc

_N = 10000
_E = 320000
_EP = 327680
_D = 128
_NB = 8
_HID = 64
_SQRT2 = math.sqrt(2.0)
_SQRT3 = math.sqrt(3.0)
_INV_SQRT_AVG = 1.0 / math.sqrt(32.0)

_NC = 2
_NS = 16
_NW = _NC * _NS


_GC = 80
_G_PER_W = _EP // _NW
_G_CHUNKS = _G_PER_W // _GC
_RING = 8


def _gather_body(feats_hbm, senders_hbm, out_hbm, idx_all, rows, sem_g, sem_w):
    c = lax.axis_index("c")
    s = lax.axis_index("s")
    wid = s * _NC + c
    base = wid * _G_PER_W

    pltpu.sync_copy(senders_hbm.at[pl.ds(base, _G_PER_W)], idx_all)

    def start_gather(i, b):
        pltpu.async_copy(
            feats_hbm.at[idx_all.at[pl.ds(i * _GC, _GC)]], rows[b], sem_g[b])

    def wait_gather(b):
        pltpu.make_async_copy(
            feats_hbm.at[idx_all.at[pl.ds(0, _GC)]], rows[b], sem_g[b]).wait()

    def start_writeout(i, b):
        pltpu.async_copy(
            rows[b], out_hbm.at[pl.ds(base + i * _GC, _GC)], sem_w[b])

    def wait_writeout(b):
        pltpu.make_async_copy(
            rows[b], out_hbm.at[pl.ds(base, _GC)], sem_w[b]).wait()

    start_gather(0, 0)

    def outer(j, _):
        for b in range(_RING):
            i = j * _RING + b
            nb = (b + 1) % _RING

            @pl.when(i >= _RING - 1)
            def _():
                wait_writeout(nb)

            @pl.when(i + 1 < _G_CHUNKS)
            def _():
                start_gather(i + 1, nb)

            wait_gather(b)
            start_writeout(i, b)
        return ()

    lax.fori_loop(0, _G_CHUNKS // _RING, outer, ())
    for b in range(1, _RING):
        wait_writeout(b)


@jax.jit
def _sc_gather(node_feats, senders):
    return pl.kernel(
        _gather_body,
        out_type=jax.ShapeDtypeStruct((_EP, _D), jnp.float32),
        mesh=plsc.VectorSubcoreMesh(core_axis_name="c", subcore_axis_name="s"),
        scratch_types=[
            pltpu.VMEM((_G_PER_W,), jnp.int32),
            [pltpu.VMEM((_GC, _D), jnp.float32) for _ in range(_RING)],
            [pltpu.SemaphoreType.DMA for _ in range(_RING)],
            [pltpu.SemaphoreType.DMA for _ in range(_RING)],
        ],
    )(node_feats, senders)


_BE = 2048
_BR = _BE // 128

_COSPI = (1.0, -4.934802200544679, 4.058712126416768, -1.3352627688545895,
          0.23533063035889327, -0.025806891390014925, 0.0019295743094039554)
_SINPI = (3.141592653589793, -5.16771278004997, 2.550164039877345,
          -0.5992645293207921, 0.08214588661112823, -0.007370430945714351,
          0.00046630280576761256)


def _poly(u, coefs):
    acc = jnp.full_like(u, coefs[-1])
    for cc in coefs[-2::-1]:
        acc = acc * u + cc
    return acc


def _msg_body(vx_ref, vy_ref, vz_ref, feat_ref, w1_ref, w2_ref, w3_ref,
              out_ref):
    vx = vx_ref[...]
    vy = vy_ref[...]
    vz = vz_ref[...]
    len2 = vx * vx + vy * vy + vz * vz
    x = jnp.sqrt(len2)
    is_zero = x == 0.0
    x_safe = jnp.where(is_zero, 1.0, x)
    inv_x = 1.0 / x_safe
    xc = jnp.minimum(x, 1.0)
    t = xc - 0.5
    u = t * t
    s1 = _poly(u, _COSPI)
    c1 = -t * _poly(u, _SINPI)
    e2 = xc * xc
    e4 = e2 * e2
    e6 = e4 * e2
    env = 1.0 - 28.0 * e6 + 48.0 * e6 * xc - 21.0 * e4 * e4
    scale = jnp.where(is_zero, 0.0, _SQRT2 * env * inv_x)
    twoc = 2.0 * c1
    sk_m1 = jnp.zeros_like(s1)
    sk = s1
    rows = []
    for _ in range(_NB):
        rows.append((sk * scale).reshape(1, _BE))
        sk, sk_m1 = twoc * sk - sk_m1, sk
    y1s = _SQRT3 * inv_x
    rows.append((vx * y1s).reshape(1, _BE))
    rows.append((vy * y1s).reshape(1, _BE))
    rows.append((vz * y1s).reshape(1, _BE))
    bundle = jnp.concatenate(rows, axis=0)
    tb = bundle.T
    radial = tb[:, :_NB]
    y1 = tb[:, _NB:]
    inv_s8 = 1.0 / math.sqrt(8.0)
    hp = jax.lax.Precision.DEFAULT
    h = jnp.dot(radial, w1_ref[...], precision=hp) * inv_s8
    h = h * jax.nn.sigmoid(h)
    h = jnp.dot(h, w2_ref[...], precision=hp) * 0.125
    h = h * jax.nn.sigmoid(h)
    mix = jnp.dot(h, w3_ref[...], precision=hp) * (0.125 * _INV_SQRT_AVG)
    feat = feat_ref[...]
    ms = feat * mix[:, :_D]
    mv = feat * mix[:, _D:]
    out_ref[0, :, :] = ms
    out_ref[1, :, :] = mv * y1[:, 0:1]
    out_ref[2, :, :] = mv * y1[:, 1:2]
    out_ref[3, :, :] = mv * y1[:, 2:3]


@jax.jit
def _messages(vx2, vy2, vz2, feat_g, W1, W2, W3):
    return pl.pallas_call(
        _msg_body,
        grid=(_EP // _BE,),
        in_specs=[
            pl.BlockSpec((_BR, 128), lambda i: (i, 0)),
            pl.BlockSpec((_BR, 128), lambda i: (i, 0)),
            pl.BlockSpec((_BR, 128), lambda i: (i, 0)),
            pl.BlockSpec((_BE, _D), lambda i: (i, 0)),
            pl.BlockSpec((_NB, _HID), lambda i: (0, 0)),
            pl.BlockSpec((_HID, _HID), lambda i: (0, 0)),
            pl.BlockSpec((_HID, 2 * _D), lambda i: (0, 0)),
        ],
        out_specs=pl.BlockSpec((4, _BE, _D), lambda i: (0, i, 0)),
        out_shape=jax.ShapeDtypeStruct((4, _EP, _D), jnp.float32),
    )(vx2, vy2, vz2, feat_g, W1, W2, W3)


_SCK = 128
_S_PER_W = _EP // _NS
_S_CHUNKS = _S_PER_W // _SCK
_N_STRIPE = 624
_N_TAIL = _N - 15 * _N_STRIPE
_SRING = 2


def _scatter_body(msg_hbm, recv_hbm, zeros_hbm, out_hbm,
                  ridx, rows, acc_sh, sem_l):
    c = lax.axis_index("c")
    s = lax.axis_index("s")
    ebase = s * _S_PER_W
    nbase = s * _N_STRIPE

    def start_loads(g, i, b):
        off = ebase + i * _SCK
        pltpu.async_copy(recv_hbm.at[pl.ds(off, _SCK)], ridx[b], sem_l[b])
        pltpu.async_copy(msg_hbm.at[pl.ds(g * _EP + off, _SCK)], rows[b],
                         sem_l[b])

    def wait_loads(b):
        pltpu.make_async_copy(
            recv_hbm.at[pl.ds(0, _SCK)], ridx[b], sem_l[b]).wait()
        pltpu.make_async_copy(
            msg_hbm.at[pl.ds(0, _SCK)], rows[b], sem_l[b]).wait()

    def do_scatter(b):
        pltpu.sync_copy(rows[b], acc_sh.at[ridx[b]], add=True)

    def one_group(g):
        pltpu.sync_copy(zeros_hbm.at[pl.ds(0, _N_STRIPE)],
                        acc_sh.at[pl.ds(nbase, _N_STRIPE)])

        @pl.when(s == _NS - 1)
        def _():
            tail = _N_TAIL - _N_STRIPE
            pltpu.sync_copy(zeros_hbm.at[pl.ds(_N_STRIPE, tail)],
                            acc_sh.at[pl.ds(15 * _N_STRIPE + _N_STRIPE, tail)])

        plsc.subcore_barrier()

        start_loads(g, 0, 0)

        def outer(j, _):
            for b in range(_SRING):
                i = j * _SRING + b
                nb = (b + 1) % _SRING

                @pl.when(i + 1 < _S_CHUNKS)
                def _():
                    start_loads(g, i + 1, nb)

                wait_loads(b)
                do_scatter(b)
            return ()

        lax.fori_loop(0, _S_CHUNKS // _SRING, outer, ())
        plsc.subcore_barrier()
        pltpu.sync_copy(acc_sh.at[pl.ds(nbase, _N_STRIPE)],
                        out_hbm.at[pl.ds(g * _N + nbase, _N_STRIPE)])

        @pl.when(s == _NS - 1)
        def _():
            tail = _N_TAIL - _N_STRIPE
            pltpu.sync_copy(
                acc_sh.at[pl.ds(16 * _N_STRIPE, tail)],
                out_hbm.at[pl.ds(g * _N + 16 * _N_STRIPE, tail)])

        plsc.subcore_barrier()

    one_group(c * 2)
    one_group(c * 2 + 1)


@jax.jit
def _sc_scatter(msg4, receivers, zeros_block):
    msg_flat = msg4.reshape(4 * _EP, _D)
    out = pl.kernel(
        _scatter_body,
        out_type=jax.ShapeDtypeStruct((4 * _N, _D), jnp.float32),
        mesh=plsc.VectorSubcoreMesh(core_axis_name="c", subcore_axis_name="s"),
        scratch_types=[
            [pltpu.VMEM((_SCK,), jnp.int32) for _ in range(_SRING)],
            [pltpu.VMEM((_SCK, _D), jnp.float32) for _ in range(_SRING)],
            pltpu.VMEM_SHARED((_N, _D), jnp.float32),
            [pltpu.SemaphoreType.DMA for _ in range(_SRING)],
        ],
    )(msg_flat, receivers, zeros_block)
    return out.reshape(4, _N, _D)


def kernel(vectors, node_feats, senders, receivers, W1, W2, W3):
    N, d = node_feats.shape
    pad = _EP - _E
    vp = jnp.pad(vectors, ((0, pad), (0, 0)))
    vx2 = vp[:, 0].reshape(_EP // 128, 128)
    vy2 = vp[:, 1].reshape(_EP // 128, 128)
    vz2 = vp[:, 2].reshape(_EP // 128, 128)
    senders_p = jnp.pad(senders.astype(jnp.int32), (0, pad))
    receivers_p = jnp.pad(receivers.astype(jnp.int32), (0, pad))
    feat_g = _sc_gather(node_feats, senders_p)
    msg4 = _messages(vx2, vy2, vz2, feat_g, W1, W2, W3)
    zeros_block = jnp.zeros((_N_TAIL, _D), jnp.float32)
    out4 = _sc_scatter(msg4, receivers_p, zeros_block)
    out_s = out4[0]
    out_v = out4[1:].transpose(1, 2, 0).reshape(N, 3 * d)
    return jnp.concatenate([out_s, out_v], axis=1)


# speedup vs baseline: 1.5008x; 1.5008x over previous
import functools
import math

import jax
import jax.numpy as jnp
from jax import lax
from jax.experimental import pallas as pl
from jax.experimental.pallas import tpu as pltpu
from jax.experimental.pallas import tpu_sc as plsc

_N = 10000
_E = 320000
_EP = 327680
_D = 128
_NB = 8
_HID = 64
_SQRT2 = math.sqrt(2.0)
_SQRT3 = math.sqrt(3.0)
_INV_SQRT_AVG = 1.0 / math.sqrt(32.0)

_NC = 2
_NS = 16
_NW = _NC * _NS


_GC = 80
_G_PER_W = _EP // _NW
_G_CHUNKS = _G_PER_W // _GC
_RING = 2


def _gather_body(feats_hbm, senders_hbm, out_hbm, idx_all, rows, table_sh,
                 sem_g, sem_w):
    c = lax.axis_index("c")
    s = lax.axis_index("s")
    wid = s * _NC + c
    base = wid * _G_PER_W

    nbase = s * _N_STRIPE
    pltpu.sync_copy(feats_hbm.at[pl.ds(nbase, _N_STRIPE)],
                    table_sh.at[pl.ds(nbase, _N_STRIPE)])

    @pl.when(s == _NS - 1)
    def _():
        tail = _N_TAIL - _N_STRIPE
        pltpu.sync_copy(feats_hbm.at[pl.ds(16 * _N_STRIPE, tail)],
                        table_sh.at[pl.ds(16 * _N_STRIPE, tail)])

    pltpu.sync_copy(senders_hbm.at[pl.ds(base, _G_PER_W)], idx_all)
    plsc.subcore_barrier()

    def start_gather(i, b):
        pltpu.async_copy(
            table_sh.at[idx_all.at[pl.ds(i * _GC, _GC)]], rows[b], sem_g[b])

    def wait_gather(b):
        pltpu.make_async_copy(
            table_sh.at[idx_all.at[pl.ds(0, _GC)]], rows[b], sem_g[b]).wait()

    def start_writeout(i, b):
        pltpu.async_copy(
            rows[b], out_hbm.at[pl.ds(base + i * _GC, _GC)], sem_w[b])

    def wait_writeout(b):
        pltpu.make_async_copy(
            rows[b], out_hbm.at[pl.ds(base, _GC)], sem_w[b]).wait()

    start_gather(0, 0)

    def outer(j, _):
        for b in range(_RING):
            i = j * _RING + b
            nb = (b + 1) % _RING

            @pl.when(i >= _RING - 1)
            def _():
                wait_writeout(nb)

            @pl.when(i + 1 < _G_CHUNKS)
            def _():
                start_gather(i + 1, nb)

            wait_gather(b)
            start_writeout(i, b)
        return ()

    lax.fori_loop(0, _G_CHUNKS // _RING, outer, ())
    for b in range(1, _RING):
        wait_writeout(b)


@jax.jit
def _sc_gather(node_feats, senders):
    return pl.kernel(
        _gather_body,
        out_type=jax.ShapeDtypeStruct((_EP, _D), jnp.float32),
        mesh=plsc.VectorSubcoreMesh(core_axis_name="c", subcore_axis_name="s"),
        scratch_types=[
            pltpu.VMEM((_G_PER_W,), jnp.int32),
            [pltpu.VMEM((_GC, _D), jnp.float32) for _ in range(_RING)],
            pltpu.VMEM_SHARED((_N, _D), jnp.float32),
            [pltpu.SemaphoreType.DMA for _ in range(_RING)],
            [pltpu.SemaphoreType.DMA for _ in range(_RING)],
        ],
    )(node_feats, senders)


_BE = 2048
_BR = _BE // 128

_COSPI = (1.0, -4.934802200544679, 4.058712126416768, -1.3352627688545895,
          0.23533063035889327, -0.025806891390014925, 0.0019295743094039554)
_SINPI = (3.141592653589793, -5.16771278004997, 2.550164039877345,
          -0.5992645293207921, 0.08214588661112823, -0.007370430945714351,
          0.00046630280576761256)


def _poly(u, coefs):
    acc = jnp.full_like(u, coefs[-1])
    for cc in coefs[-2::-1]:
        acc = acc * u + cc
    return acc


def _msg_body(vx_ref, vy_ref, vz_ref, feat_ref, w1_ref, w2_ref, w3_ref,
              out_ref):
    vx = vx_ref[...]
    vy = vy_ref[...]
    vz = vz_ref[...]
    len2 = vx * vx + vy * vy + vz * vz
    x = jnp.sqrt(len2)
    is_zero = x == 0.0
    x_safe = jnp.where(is_zero, 1.0, x)
    inv_x = 1.0 / x_safe
    xc = jnp.minimum(x, 1.0)
    t = xc - 0.5
    u = t * t
    s1 = _poly(u, _COSPI)
    c1 = -t * _poly(u, _SINPI)
    e2 = xc * xc
    e4 = e2 * e2
    e6 = e4 * e2
    env = 1.0 - 28.0 * e6 + 48.0 * e6 * xc - 21.0 * e4 * e4
    scale = jnp.where(is_zero, 0.0, _SQRT2 * env * inv_x)
    twoc = 2.0 * c1
    sk_m1 = jnp.zeros_like(s1)
    sk = s1
    rows = []
    for _ in range(_NB):
        rows.append((sk * scale).reshape(1, _BE))
        sk, sk_m1 = twoc * sk - sk_m1, sk
    y1s = _SQRT3 * inv_x
    rows.append((vx * y1s).reshape(1, _BE))
    rows.append((vy * y1s).reshape(1, _BE))
    rows.append((vz * y1s).reshape(1, _BE))
    bundle = jnp.concatenate(rows, axis=0)
    tb = bundle.T
    radial = tb[:, :_NB]
    y1 = tb[:, _NB:]
    inv_s8 = 1.0 / math.sqrt(8.0)
    hp = jax.lax.Precision.DEFAULT
    h = jnp.dot(radial, w1_ref[...], precision=hp) * inv_s8
    h = h * jax.nn.sigmoid(h)
    h = jnp.dot(h, w2_ref[...], precision=hp) * 0.125
    h = h * jax.nn.sigmoid(h)
    mix = jnp.dot(h, w3_ref[...], precision=hp) * (0.125 * _INV_SQRT_AVG)
    feat = feat_ref[...]
    ms = feat * mix[:, :_D]
    mv = feat * mix[:, _D:]
    out_ref[0, :, :] = ms
    out_ref[1, :, :] = mv * y1[:, 0:1]
    out_ref[2, :, :] = mv * y1[:, 1:2]
    out_ref[3, :, :] = mv * y1[:, 2:3]


@jax.jit
def _messages(vx2, vy2, vz2, feat_g, W1, W2, W3):
    return pl.pallas_call(
        _msg_body,
        grid=(_EP // _BE,),
        in_specs=[
            pl.BlockSpec((_BR, 128), lambda i: (i, 0)),
            pl.BlockSpec((_BR, 128), lambda i: (i, 0)),
            pl.BlockSpec((_BR, 128), lambda i: (i, 0)),
            pl.BlockSpec((_BE, _D), lambda i: (i, 0)),
            pl.BlockSpec((_NB, _HID), lambda i: (0, 0)),
            pl.BlockSpec((_HID, _HID), lambda i: (0, 0)),
            pl.BlockSpec((_HID, 2 * _D), lambda i: (0, 0)),
        ],
        out_specs=pl.BlockSpec((4, _BE, _D), lambda i: (0, i, 0)),
        out_shape=jax.ShapeDtypeStruct((4, _EP, _D), jnp.float32),
    )(vx2, vy2, vz2, feat_g, W1, W2, W3)


_SCK = 128
_S_PER_W = _EP // _NS
_S_CHUNKS = _S_PER_W // _SCK
_N_STRIPE = 624
_N_TAIL = _N - 15 * _N_STRIPE
_SRING = 2


def _scatter_body(msg_hbm, recv_hbm, zeros_hbm, out_hbm,
                  ridx, rows, acc_sh, sem_l):
    c = lax.axis_index("c")
    s = lax.axis_index("s")
    ebase = s * _S_PER_W
    nbase = s * _N_STRIPE

    def start_loads(g, i, b):
        off = ebase + i * _SCK
        pltpu.async_copy(recv_hbm.at[pl.ds(off, _SCK)], ridx[b], sem_l[b])
        pltpu.async_copy(msg_hbm.at[pl.ds(g * _EP + off, _SCK)], rows[b],
                         sem_l[b])

    def wait_loads(b):
        pltpu.make_async_copy(
            recv_hbm.at[pl.ds(0, _SCK)], ridx[b], sem_l[b]).wait()
        pltpu.make_async_copy(
            msg_hbm.at[pl.ds(0, _SCK)], rows[b], sem_l[b]).wait()

    def do_scatter(b):
        pltpu.sync_copy(rows[b], acc_sh.at[ridx[b]], add=True)

    def one_group(g):
        pltpu.sync_copy(zeros_hbm.at[pl.ds(0, _N_STRIPE)],
                        acc_sh.at[pl.ds(nbase, _N_STRIPE)])

        @pl.when(s == _NS - 1)
        def _():
            tail = _N_TAIL - _N_STRIPE
            pltpu.sync_copy(zeros_hbm.at[pl.ds(_N_STRIPE, tail)],
                            acc_sh.at[pl.ds(15 * _N_STRIPE + _N_STRIPE, tail)])

        plsc.subcore_barrier()

        start_loads(g, 0, 0)

        def outer(j, _):
            for b in range(_SRING):
                i = j * _SRING + b
                nb = (b + 1) % _SRING

                @pl.when(i + 1 < _S_CHUNKS)
                def _():
                    start_loads(g, i + 1, nb)

                wait_loads(b)
                do_scatter(b)
            return ()

        lax.fori_loop(0, _S_CHUNKS // _SRING, outer, ())
        plsc.subcore_barrier()
        pltpu.sync_copy(acc_sh.at[pl.ds(nbase, _N_STRIPE)],
                        out_hbm.at[pl.ds(g * _N + nbase, _N_STRIPE)])

        @pl.when(s == _NS - 1)
        def _():
            tail = _N_TAIL - _N_STRIPE
            pltpu.sync_copy(
                acc_sh.at[pl.ds(16 * _N_STRIPE, tail)],
                out_hbm.at[pl.ds(g * _N + 16 * _N_STRIPE, tail)])

        plsc.subcore_barrier()

    one_group(c * 2)
    one_group(c * 2 + 1)


@jax.jit
def _sc_scatter(msg4, receivers, zeros_block):
    msg_flat = msg4.reshape(4 * _EP, _D)
    out = pl.kernel(
        _scatter_body,
        out_type=jax.ShapeDtypeStruct((4 * _N, _D), jnp.float32),
        mesh=plsc.VectorSubcoreMesh(core_axis_name="c", subcore_axis_name="s"),
        scratch_types=[
            [pltpu.VMEM((_SCK,), jnp.int32) for _ in range(_SRING)],
            [pltpu.VMEM((_SCK, _D), jnp.float32) for _ in range(_SRING)],
            pltpu.VMEM_SHARED((_N, _D), jnp.float32),
            [pltpu.SemaphoreType.DMA for _ in range(_SRING)],
        ],
    )(msg_flat, receivers, zeros_block)
    return out.reshape(4, _N, _D)


def kernel(vectors, node_feats, senders, receivers, W1, W2, W3):
    N, d = node_feats.shape
    pad = _EP - _E
    vp = jnp.pad(vectors, ((0, pad), (0, 0)))
    vx2 = vp[:, 0].reshape(_EP // 128, 128)
    vy2 = vp[:, 1].reshape(_EP // 128, 128)
    vz2 = vp[:, 2].reshape(_EP // 128, 128)
    senders_p = jnp.pad(senders.astype(jnp.int32), (0, pad))
    receivers_p = jnp.pad(receivers.astype(jnp.int32), (0, pad))
    feat_g = _sc_gather(node_feats, senders_p)
    msg4 = _messages(vx2, vy2, vz2, feat_g, W1, W2, W3)
    zeros_block = jnp.zeros((_N_TAIL, _D), jnp.float32)
    out4 = _sc_scatter(msg4, receivers_p, zeros_block)
    out_s = out4[0]
    out_v = out4[1:].transpose(1, 2, 0).reshape(N, 3 * d)
    return jnp.concatenate([out_s, out_v], axis=1)


# speedup vs baseline: 1.5017x; 1.0007x over previous
import math

import jax
import jax.numpy as jnp
from jax import lax
from jax.experimental import pallas as pl
from jax.experimental.pallas import tpu as pltpu
from jax.experimental.pallas import tpu_sc as plsc

_N = 10000
_E = 320000
_EP = 327680
_D = 128
_NB = 8
_HID = 64
_SQRT2 = math.sqrt(2.0)
_SQRT3 = math.sqrt(3.0)
_INV_SQRT_AVG = 1.0 / math.sqrt(32.0)

_NC = 2
_NS = 16
_NW = _NC * _NS


_GC = 80
_G_PER_W = _EP // _NW
_G_CHUNKS = _G_PER_W // _GC
_RING = 2


def _gather_body(feats_hbm, senders_hbm, out_hbm, idx_all, rows, table_sh,
                 sem_g, sem_w):
    c = lax.axis_index("c")
    s = lax.axis_index("s")
    wid = s * _NC + c
    base = wid * _G_PER_W

    nbase = s * _N_STRIPE
    pltpu.sync_copy(feats_hbm.at[pl.ds(nbase, _N_STRIPE)],
                    table_sh.at[pl.ds(nbase, _N_STRIPE)])

    @pl.when(s == _NS - 1)
    def _():
        tail = _N_TAIL - _N_STRIPE
        pltpu.sync_copy(feats_hbm.at[pl.ds(16 * _N_STRIPE, tail)],
                        table_sh.at[pl.ds(16 * _N_STRIPE, tail)])

    pltpu.sync_copy(senders_hbm.at[pl.ds(base, _G_PER_W)], idx_all)
    plsc.subcore_barrier()

    def start_gather(i, b):
        pltpu.async_copy(
            table_sh.at[idx_all.at[pl.ds(i * _GC, _GC)]], rows[b], sem_g[b])

    def wait_gather(b):
        pltpu.make_async_copy(
            table_sh.at[idx_all.at[pl.ds(0, _GC)]], rows[b], sem_g[b]).wait()

    def start_writeout(i, b):
        pltpu.async_copy(
            rows[b], out_hbm.at[pl.ds(base + i * _GC, _GC)], sem_w[b])

    def wait_writeout(b):
        pltpu.make_async_copy(
            rows[b], out_hbm.at[pl.ds(base, _GC)], sem_w[b]).wait()

    start_gather(0, 0)

    def outer(j, _):
        for b in range(_RING):
            i = j * _RING + b
            nb = (b + 1) % _RING

            @pl.when(i >= _RING - 1)
            def _():
                wait_writeout(nb)

            @pl.when(i + 1 < _G_CHUNKS)
            def _():
                start_gather(i + 1, nb)

            wait_gather(b)
            start_writeout(i, b)
        return ()

    lax.fori_loop(0, _G_CHUNKS // _RING, outer, ())
    for b in range(1, _RING):
        wait_writeout(b)


@jax.jit
def _sc_gather(node_feats, senders):
    return pl.kernel(
        _gather_body,
        out_type=jax.ShapeDtypeStruct((_EP, _D), jnp.float32),
        mesh=plsc.VectorSubcoreMesh(core_axis_name="c", subcore_axis_name="s"),
        scratch_types=[
            pltpu.VMEM((_G_PER_W,), jnp.int32),
            [pltpu.VMEM((_GC, _D), jnp.float32) for _ in range(_RING)],
            pltpu.VMEM_SHARED((_N, _D), jnp.float32),
            [pltpu.SemaphoreType.DMA for _ in range(_RING)],
            [pltpu.SemaphoreType.DMA for _ in range(_RING)],
        ],
    )(node_feats, senders)


_BE = 2048
_BR = _BE // 128

_COSPI = (1.0, -4.934802200544679, 4.058712126416768, -1.3352627688545895,
          0.23533063035889327, -0.025806891390014925, 0.0019295743094039554)
_SINPI = (3.141592653589793, -5.16771278004997, 2.550164039877345,
          -0.5992645293207921, 0.08214588661112823, -0.007370430945714351,
          0.00046630280576761256)


def _poly(u, coefs):
    acc = jnp.full_like(u, coefs[-1])
    for cc in coefs[-2::-1]:
        acc = acc * u + cc
    return acc


def _msg_body(vx_ref, vy_ref, vz_ref, feat_ref, w1_ref, w2_ref, w3_ref,
              out_ref):
    vx = vx_ref[...]
    vy = vy_ref[...]
    vz = vz_ref[...]
    len2 = vx * vx + vy * vy + vz * vz
    x = jnp.sqrt(len2)
    is_zero = x == 0.0
    x_safe = jnp.where(is_zero, 1.0, x)
    inv_x = 1.0 / x_safe
    xc = jnp.minimum(x, 1.0)
    t = xc - 0.5
    u = t * t
    s1 = _poly(u, _COSPI)
    c1 = -t * _poly(u, _SINPI)
    e2 = xc * xc
    e4 = e2 * e2
    e6 = e4 * e2
    env = 1.0 - 28.0 * e6 + 48.0 * e6 * xc - 21.0 * e4 * e4
    scale = jnp.where(is_zero, 0.0, _SQRT2 * env * inv_x)
    twoc = 2.0 * c1
    sk_m1 = jnp.zeros_like(s1)
    sk = s1
    rows = []
    for _ in range(_NB):
        rows.append((sk * scale).reshape(1, _BE))
        sk, sk_m1 = twoc * sk - sk_m1, sk
    y1s = _SQRT3 * inv_x
    rows.append((vx * y1s).reshape(1, _BE))
    rows.append((vy * y1s).reshape(1, _BE))
    rows.append((vz * y1s).reshape(1, _BE))
    bundle = jnp.concatenate(rows, axis=0)
    tb = bundle.T
    radial = tb[:, :_NB]
    y1 = tb[:, _NB:]
    inv_s8 = 1.0 / math.sqrt(8.0)
    hp = jax.lax.Precision.DEFAULT
    h = jnp.dot(radial, w1_ref[...], precision=hp) * inv_s8
    h = h * jax.nn.sigmoid(h)
    h = jnp.dot(h, w2_ref[...], precision=hp) * 0.125
    h = h * jax.nn.sigmoid(h)
    mix = jnp.dot(h, w3_ref[...], precision=hp) * (0.125 * _INV_SQRT_AVG)
    feat = feat_ref[...]
    ms = feat * mix[:, :_D]
    mv = feat * mix[:, _D:]
    out_ref[0, :, :] = ms
    out_ref[1, :, :] = mv * y1[:, 0:1]
    out_ref[2, :, :] = mv * y1[:, 1:2]
    out_ref[3, :, :] = mv * y1[:, 2:3]


@jax.jit
def _messages(vx2, vy2, vz2, feat_g, W1, W2, W3):
    return pl.pallas_call(
        _msg_body,
        grid=(_EP // _BE,),
        in_specs=[
            pl.BlockSpec((_BR, 128), lambda i: (i, 0)),
            pl.BlockSpec((_BR, 128), lambda i: (i, 0)),
            pl.BlockSpec((_BR, 128), lambda i: (i, 0)),
            pl.BlockSpec((_BE, _D), lambda i: (i, 0)),
            pl.BlockSpec((_NB, _HID), lambda i: (0, 0)),
            pl.BlockSpec((_HID, _HID), lambda i: (0, 0)),
            pl.BlockSpec((_HID, 2 * _D), lambda i: (0, 0)),
        ],
        out_specs=pl.BlockSpec((4, _BE, _D), lambda i: (0, i, 0)),
        out_shape=jax.ShapeDtypeStruct((4, _EP, _D), jnp.float32),
    )(vx2, vy2, vz2, feat_g, W1, W2, W3)


_SCK = 128
_S_PER_W = _EP // _NS
_S_CHUNKS = _S_PER_W // _SCK
_N_STRIPE = 624
_N_TAIL = _N - 15 * _N_STRIPE
_SRING = 2


def _scatter_body(msg_hbm, recv_hbm, zeros_hbm, out_hbm,
                  ridx, rows, acc_sh, sem_l):
    c = lax.axis_index("c")
    s = lax.axis_index("s")
    ebase = s * _S_PER_W
    nbase = s * _N_STRIPE

    def start_loads(g, i, b):
        off = ebase + i * _SCK
        pltpu.async_copy(recv_hbm.at[pl.ds(off, _SCK)], ridx[b], sem_l[b])
        pltpu.async_copy(msg_hbm.at[pl.ds(g * _EP + off, _SCK)], rows[b],
                         sem_l[b])

    def wait_loads(b):
        pltpu.make_async_copy(
            recv_hbm.at[pl.ds(0, _SCK)], ridx[b], sem_l[b]).wait()
        pltpu.make_async_copy(
            msg_hbm.at[pl.ds(0, _SCK)], rows[b], sem_l[b]).wait()

    def do_scatter(b):
        pltpu.sync_copy(rows[b], acc_sh.at[ridx[b]], add=True)

    def one_group(g):
        pltpu.sync_copy(zeros_hbm.at[pl.ds(0, _N_STRIPE)],
                        acc_sh.at[pl.ds(nbase, _N_STRIPE)])

        @pl.when(s == _NS - 1)
        def _():
            tail = _N_TAIL - _N_STRIPE
            pltpu.sync_copy(zeros_hbm.at[pl.ds(_N_STRIPE, tail)],
                            acc_sh.at[pl.ds(15 * _N_STRIPE + _N_STRIPE, tail)])

        plsc.subcore_barrier()

        start_loads(g, 0, 0)

        def outer(j, _):
            for b in range(_SRING):
                i = j * _SRING + b
                nb = (b + 1) % _SRING

                @pl.when(i + 1 < _S_CHUNKS)
                def _():
                    start_loads(g, i + 1, nb)

                wait_loads(b)
                do_scatter(b)
            return ()

        lax.fori_loop(0, _S_CHUNKS // _SRING, outer, ())
        plsc.subcore_barrier()
        pltpu.sync_copy(acc_sh.at[pl.ds(nbase, _N_STRIPE)],
                        out_hbm.at[pl.ds(g * _N + nbase, _N_STRIPE)])

        @pl.when(s == _NS - 1)
        def _():
            tail = _N_TAIL - _N_STRIPE
            pltpu.sync_copy(
                acc_sh.at[pl.ds(16 * _N_STRIPE, tail)],
                out_hbm.at[pl.ds(g * _N + 16 * _N_STRIPE, tail)])

        plsc.subcore_barrier()

    one_group(c * 2)
    one_group(c * 2 + 1)


@jax.jit
def _sc_scatter(msg4, receivers, zeros_block):
    msg_flat = msg4.reshape(4 * _EP, _D)
    out = pl.kernel(
        _scatter_body,
        out_type=jax.ShapeDtypeStruct((4 * _N, _D), jnp.float32),
        mesh=plsc.VectorSubcoreMesh(core_axis_name="c", subcore_axis_name="s"),
        scratch_types=[
            [pltpu.VMEM((_SCK,), jnp.int32) for _ in range(_SRING)],
            [pltpu.VMEM((_SCK, _D), jnp.float32) for _ in range(_SRING)],
            pltpu.VMEM_SHARED((_N, _D), jnp.float32),
            [pltpu.SemaphoreType.DMA for _ in range(_SRING)],
        ],
    )(msg_flat, receivers, zeros_block)
    return out.reshape(4, _N, _D)


def kernel(vectors, node_feats, senders, receivers, W1, W2, W3):
    N, d = node_feats.shape
    pad = _EP - _E
    vp = jnp.pad(vectors, ((0, pad), (0, 0)))
    vx2 = vp[:, 0].reshape(_EP // 128, 128)
    vy2 = vp[:, 1].reshape(_EP // 128, 128)
    vz2 = vp[:, 2].reshape(_EP // 128, 128)
    senders_p = jnp.pad(senders.astype(jnp.int32), (0, pad))
    receivers_p = jnp.pad(receivers.astype(jnp.int32), (0, pad))
    feat_g = _sc_gather(node_feats, senders_p)
    msg4 = _messages(vx2, vy2, vz2, feat_g, W1, W2, W3)
    zeros_block = jnp.zeros((_N_TAIL, _D), jnp.float32)
    out4 = _sc_scatter(msg4, receivers_p, zeros_block)
    out_s = out4[0]
    out_v = out4[1:].transpose(1, 2, 0).reshape(N, 3 * d)
    return jnp.concatenate([out_s, out_v], axis=1)
